# baseline (speedup 1.0000x reference)
import dataclasses
import functools

import jax
import jax.numpy as jnp
from jax import lax
from jax.experimental import pallas as pl
from jax.experimental.pallas import tpu as pltpu
from jax.experimental.pallas import tpu_sc as plsc

_NC = 2
_NS = 16
_NW = _NC * _NS
_L = 16
_EB = 320


def _vector_mesh():
    return plsc.VectorSubcoreMesh(
        core_axis_name="c", subcore_axis_name="s",
        num_cores=_NC, num_subcores=_NS)


def _sc_params():
    cp = pltpu.CompilerParams()
    fields = pltpu.CompilerParams.__dataclass_fields__
    if "needs_layout_passes" in fields:
        cp = dataclasses.replace(cp, needs_layout_passes=False)
    if "use_tc_tiling_on_sc" in fields:
        cp = dataclasses.replace(cp, use_tc_tiling_on_sc=False)
    return cp


def _make_deg_kernel(n, eper):
    @functools.partial(
        pl.kernel,
        out_type=jax.ShapeDtypeStruct((_NW, 1, n), jnp.float32),
        mesh=_vector_mesh(),
        compiler_params=_sc_params(),
        scratch_types=[
            pltpu.VMEM((1, eper), jnp.int32),
            pltpu.VMEM((1, eper), jnp.float32),
            pltpu.VMEM((1, n), jnp.float32),
        ],
    )
    def deg_kernel(col_hbm, ew_hbm, out_hbm, colv, ewv, hist):
        cid = lax.axis_index("c")
        sid = lax.axis_index("s")
        wid = sid * _NC + cid
        pltpu.sync_copy(col_hbm.at[wid], colv)
        pltpu.sync_copy(ew_hbm.at[wid], ewv)
        zeros = jnp.zeros((_L,), jnp.float32)
        zi = jnp.zeros((_L,), jnp.int32)

        @pl.loop(0, n, step=_L)
        def _(i):
            hist[0, pl.ds(i, _L)] = zeros

        @pl.loop(0, eper, step=_L)
        def _(j):
            idx = colv[0, pl.ds(j, _L)]
            w = ewv[0, pl.ds(j, _L)]
            plsc.addupdate_scatter(hist, [zi, idx], w)

        pltpu.sync_copy(hist, out_hbm.at[wid])

    return deg_kernel


def _make_agg_kernel(n, h, nb0, nb1, eb):
    rpt = n // _NS
    nbm = max(nb0, nb1)

    assert nb0 % 2 == 0 and nb1 % 2 == 0

    @functools.partial(
        pl.kernel,
        out_type=jax.ShapeDtypeStruct((_NC, n, h), jnp.float32),
        mesh=_vector_mesh(),
        compiler_params=_sc_params(),
        scratch_types=[
            pltpu.VMEM_SHARED((n, h), jnp.float32),
            pltpu.VMEM((nbm, eb), jnp.int32),
            pltpu.VMEM((nbm, eb), jnp.int32),
            pltpu.VMEM((nbm, eb), jnp.float32),
            pltpu.VMEM((eb, h), jnp.float32),
            pltpu.VMEM((eb, h), jnp.float32),
            pltpu.SemaphoreType.DMA,
            pltpu.SemaphoreType.DMA,
            pltpu.SemaphoreType.DMA,
            pltpu.SemaphoreType.DMA,
        ],
    )
    def agg_kernel(g_hbm, r0_hbm, c0_hbm, w0_hbm, r1_hbm, c1_hbm, w1_hbm,
                   out_hbm, acc, rowv, colv, ewv, rows0, rows1,
                   gsem0, gsem1, ssem0, ssem1):
        cid = lax.axis_index("c")
        sid = lax.axis_index("s")
        base = pl.multiple_of(sid * rpt, 8)
        pltpu.sync_copy(g_hbm.at[pl.ds(base, rpt)], acc.at[pl.ds(base, rpt)])
        plsc.subcore_barrier()

        bufs = (rows0, rows1)
        gsems = (gsem0, gsem1)
        ssems = (ssem0, ssem1)

        def start_gather(b, k):
            pltpu.async_copy(g_hbm.at[rowv.at[b]], bufs[k], gsems[k])

        def wait_gather(b, k):
            pltpu.make_async_copy(g_hbm.at[rowv.at[b]], bufs[k],
                                  gsems[k]).wait()

        def start_scatter(b, k):
            pltpu.async_copy(bufs[k], acc.at[colv.at[b]], ssems[k], add=True)

        def wait_scatter(b, k):
            pltpu.make_async_copy(bufs[k], acc.at[colv.at[b]],
                                  ssems[k]).wait()

        bcast_dn = lax.GatherDimensionNumbers(
            offset_dims=(), collapsed_slice_dims=(0,), start_index_map=(0,))

        def lane_bcast(v16, l):
            idx = jnp.full((_L, 1), l, jnp.int32)
            return lax.gather(v16, idx, bcast_dn, (1,),
                              mode=lax.GatherScatterMode.PROMISE_IN_BOUNDS)

        def scale(b, k):
            buf = bufs[k]

            @pl.loop(0, eb, step=_L)
            def _(j):
                w16 = ewv[b, pl.ds(j, _L)]
                for l in range(_L):
                    wb = lane_bcast(w16, l)
                    for f0 in range(0, h, _L):
                        sl = (j + l, pl.ds(f0, _L))
                        buf[sl] = buf[sl] * wb

        def pipeline(row_hbm, col_hbm, ew_hbm, nb):
            pltpu.sync_copy(row_hbm.at[sid], rowv.at[pl.ds(0, nb)])
            pltpu.sync_copy(col_hbm.at[sid], colv.at[pl.ds(0, nb)])
            pltpu.sync_copy(ew_hbm.at[sid], ewv.at[pl.ds(0, nb)])
            start_gather(0, 0)

            @pl.loop(0, nb, step=2)
            def _(b0):
                wait_gather(b0, 0)

                @pl.when(b0 > 0)
                def _():
                    wait_scatter(b0 - 1, 1)

                start_gather(b0 + 1, 1)
                scale(b0, 0)
                start_scatter(b0, 0)
                wait_gather(b0 + 1, 1)
                scale(b0 + 1, 1)

                @pl.when(b0 + 2 < nb)
                def _():
                    wait_scatter(b0, 0)
                    start_gather(b0 + 2, 0)

                start_scatter(b0 + 1, 1)

            wait_scatter(nb - 2, 0)
            wait_scatter(nb - 1, 1)

        @pl.when(cid == 0)
        def _():
            pipeline(r0_hbm, c0_hbm, w0_hbm, nb0)

        @pl.when(cid == 1)
        def _():
            pipeline(r1_hbm, c1_hbm, w1_hbm, nb1)

        plsc.subcore_barrier()
        pltpu.sync_copy(acc.at[pl.ds(base, rpt)],
                        out_hbm.at[cid, pl.ds(base, rpt)])

    return agg_kernel


def _tc_h2(x, w1, b1, w2):
    n, f_in = x.shape
    hdim = w1.shape[1]
    blk = 1000

    def body(x_ref, w1_ref, b1_ref, w2_ref, o_ref):
        hb = jnp.dot(x_ref[...], w1_ref[...],
                     preferred_element_type=jnp.float32) + b1_ref[...]
        hb = jnp.maximum(hb, 0.0)
        o_ref[...] = jnp.dot(hb, w2_ref[...],
                             preferred_element_type=jnp.float32)

    return pl.pallas_call(
        body,
        grid=(n // blk,),
        in_specs=[
            pl.BlockSpec((blk, f_in), lambda i: (i, 0)),
            pl.BlockSpec((f_in, hdim), lambda i: (0, 0)),
            pl.BlockSpec((1, hdim), lambda i: (0, 0)),
            pl.BlockSpec((hdim, hdim), lambda i: (0, 0)),
        ],
        out_specs=pl.BlockSpec((blk, hdim), lambda i: (i, 0)),
        out_shape=jax.ShapeDtypeStruct((n, hdim), jnp.float32),
    )(x, w1, b1.reshape(1, hdim), w2)


def _dinv_col(d_ref):
    s = jnp.sum(d_ref[...], axis=0)
    return 1.0 / jnp.sqrt(1.0 + s)


def _tc_g(deg_parts, h2):
    nw, n, _ = deg_parts.shape
    hdim = h2.shape[1]
    blk = 1000

    def body(d_ref, h_ref, o_ref):
        o_ref[...] = _dinv_col(d_ref) * h_ref[...]

    return pl.pallas_call(
        body,
        grid=(n // blk,),
        in_specs=[
            pl.BlockSpec((nw, blk, 1), lambda i: (0, i, 0)),
            pl.BlockSpec((blk, hdim), lambda i: (i, 0)),
        ],
        out_specs=pl.BlockSpec((blk, hdim), lambda i: (i, 0)),
        out_shape=jax.ShapeDtypeStruct((n, hdim), jnp.float32),
    )(deg_parts, h2)


def _tc_final(deg_parts, g, parts, b_conv, w_out, b_out):
    nw, n, _ = deg_parts.shape
    hdim = g.shape[1]
    c = w_out.shape[1]
    blk = 1000

    def body(d_ref, g_ref, p_ref, bc_ref, wo_ref, bo_ref, o_ref):
        dinv = _dinv_col(d_ref)
        agg = p_ref[0] + p_ref[1] - g_ref[...]
        hr = jnp.maximum(dinv * agg + bc_ref[...], 0.0)
        logits = jnp.dot(hr, wo_ref[...],
                         preferred_element_type=jnp.float32) + bo_ref[...]
        m = jnp.max(logits, axis=-1, keepdims=True)
        ex = jnp.exp(logits - m)
        lse = jnp.log(jnp.sum(ex, axis=-1, keepdims=True))
        o_ref[...] = (logits - m) - lse

    return pl.pallas_call(
        body,
        grid=(n // blk,),
        in_specs=[
            pl.BlockSpec((nw, blk, 1), lambda i: (0, i, 0)),
            pl.BlockSpec((blk, hdim), lambda i: (i, 0)),
            pl.BlockSpec((_NC, blk, hdim), lambda i: (0, i, 0)),
            pl.BlockSpec((1, hdim), lambda i: (0, 0)),
            pl.BlockSpec((hdim, c), lambda i: (0, 0)),
            pl.BlockSpec((1, c), lambda i: (0, 0)),
        ],
        out_specs=pl.BlockSpec((blk, c), lambda i: (i, 0)),
        out_shape=jax.ShapeDtypeStruct((n, c), jnp.float32),
    )(deg_parts, g, parts, b_conv.reshape(1, hdim), w_out,
      b_out.reshape(1, c))


def kernel(x, edge_index, edge_weight,
           W_first, b_first, W_conv, b_conv, W_out, b_out):
    n, _ = x.shape
    hdim = W_conv.shape[0]
    e = edge_weight.shape[0]
    nbt = 2 * (-(-e // (2 * _NS * _EB)))
    ep = _NS * nbt * _EB
    pad = ep - e

    row = edge_index[0]
    col = edge_index[1]
    ew = edge_weight
    if pad:
        zi = jnp.zeros((pad,), row.dtype)
        row = jnp.concatenate([row, zi])
        col = jnp.concatenate([col, zi])
        ew = jnp.concatenate([ew, jnp.zeros((pad,), ew.dtype)])

    nb0 = max(2, 2 * int(round(nbt * 0.625 / 2)))
    nb1 = nbt - nb0
    s0 = _NS * nb0 * _EB
    r0 = row[:s0].reshape(_NS, nb0, _EB)
    c0 = col[:s0].reshape(_NS, nb0, _EB)
    w0 = ew[:s0].reshape(_NS, nb0, _EB)
    r1 = row[s0:].reshape(_NS, nb1, _EB)
    c1 = col[s0:].reshape(_NS, nb1, _EB)
    w1 = ew[s0:].reshape(_NS, nb1, _EB)

    eper = ep // _NW
    deg_parts = _make_deg_kernel(n, eper)(
        col.reshape(_NW, 1, eper),
        ew.reshape(_NW, 1, eper)).reshape(_NW, n, 1)
    h2 = _tc_h2(x, W_first, b_first, W_conv)
    g = _tc_g(deg_parts, h2)
    n_pad = -(-n // (8 * _NS)) * (8 * _NS)
    g_p = (g if n_pad == n else
           jnp.concatenate([g, jnp.zeros((n_pad - n, hdim), g.dtype)]))
    parts = _make_agg_kernel(n_pad, hdim, nb0, nb1, _EB)(
        g_p, r0, c0, w0, r1, c1, w1)
    return _tc_final(deg_parts, g, parts, b_conv, W_out, b_out)


# speedup vs baseline: 1.6224x; 1.6224x over previous
import dataclasses
import functools

import jax
import jax.numpy as jnp
from jax import lax
from jax.experimental import pallas as pl
from jax.experimental.pallas import tpu as pltpu
from jax.experimental.pallas import tpu_sc as plsc

_NC = 2
_NS = 16
_NW = _NC * _NS
_L = 16
_EB = 320


def _vector_mesh():
    return plsc.VectorSubcoreMesh(
        core_axis_name="c", subcore_axis_name="s",
        num_cores=_NC, num_subcores=_NS)


def _sc_params():
    cp = pltpu.CompilerParams()
    fields = pltpu.CompilerParams.__dataclass_fields__
    if "needs_layout_passes" in fields:
        cp = dataclasses.replace(cp, needs_layout_passes=False)
    if "use_tc_tiling_on_sc" in fields:
        cp = dataclasses.replace(cp, use_tc_tiling_on_sc=False)
    return cp


def _make_deg_kernel(n, eper):
    @functools.partial(
        pl.kernel,
        out_type=jax.ShapeDtypeStruct((_NW, 1, n), jnp.float32),
        mesh=_vector_mesh(),
        compiler_params=_sc_params(),
        scratch_types=[
            pltpu.VMEM((1, eper), jnp.int32),
            pltpu.VMEM((1, eper), jnp.float32),
            pltpu.VMEM((1, n), jnp.float32),
        ],
    )
    def deg_kernel(col_hbm, ew_hbm, out_hbm, colv, ewv, hist):
        cid = lax.axis_index("c")
        sid = lax.axis_index("s")
        wid = sid * _NC + cid
        pltpu.sync_copy(col_hbm.at[wid], colv)
        pltpu.sync_copy(ew_hbm.at[wid], ewv)
        zeros = jnp.zeros((_L,), jnp.float32)
        zi = jnp.zeros((_L,), jnp.int32)

        @pl.loop(0, n, step=_L)
        def _(i):
            hist[0, pl.ds(i, _L)] = zeros

        @pl.loop(0, eper, step=_L)
        def _(j):
            idx = colv[0, pl.ds(j, _L)]
            w = ewv[0, pl.ds(j, _L)]
            plsc.addupdate_scatter(hist, [zi, idx], w)

        pltpu.sync_copy(hist, out_hbm.at[wid])

    return deg_kernel


def _make_agg_kernel(n, h, nb0, nb1, eb):
    rpt = n // _NS
    nbm = max(nb0, nb1)

    assert nb0 % 2 == 0 and nb1 % 2 == 0

    @functools.partial(
        pl.kernel,
        out_type=jax.ShapeDtypeStruct((_NC, n, h), jnp.float32),
        mesh=_vector_mesh(),
        compiler_params=_sc_params(),
        scratch_types=[
            pltpu.VMEM_SHARED((n, h), jnp.float32),
            pltpu.VMEM((nbm, eb), jnp.int32),
            pltpu.VMEM((nbm, eb), jnp.int32),
            pltpu.VMEM((nbm, eb), jnp.float32),
            pltpu.VMEM((eb, h), jnp.float32),
            pltpu.VMEM((eb, h), jnp.float32),
            pltpu.SemaphoreType.DMA,
            pltpu.SemaphoreType.DMA,
            pltpu.SemaphoreType.DMA,
            pltpu.SemaphoreType.DMA,
        ],
    )
    def agg_kernel(g_hbm, r0_hbm, c0_hbm, w0_hbm, r1_hbm, c1_hbm, w1_hbm,
                   out_hbm, acc, rowv, colv, ewv, rows0, rows1,
                   gsem0, gsem1, ssem0, ssem1):
        cid = lax.axis_index("c")
        sid = lax.axis_index("s")
        base = pl.multiple_of(sid * rpt, 8)
        pltpu.sync_copy(g_hbm.at[pl.ds(base, rpt)], acc.at[pl.ds(base, rpt)])
        plsc.subcore_barrier()

        bufs = (rows0, rows1)
        gsems = (gsem0, gsem1)
        ssems = (ssem0, ssem1)

        def start_gather(b, k):
            pltpu.async_copy(g_hbm.at[rowv.at[b]], bufs[k], gsems[k])

        def wait_gather(b, k):
            pltpu.make_async_copy(g_hbm.at[rowv.at[b]], bufs[k],
                                  gsems[k]).wait()

        def start_scatter(b, k):
            pltpu.async_copy(bufs[k], acc.at[colv.at[b]], ssems[k], add=True)

        def wait_scatter(b, k):
            pltpu.make_async_copy(bufs[k], acc.at[colv.at[b]],
                                  ssems[k]).wait()

        bcast_dn = lax.GatherDimensionNumbers(
            offset_dims=(), collapsed_slice_dims=(0,), start_index_map=(0,))

        def lane_bcast(v16, l):
            idx = jnp.full((_L, 1), l, jnp.int32)
            return lax.gather(v16, idx, bcast_dn, (1,),
                              mode=lax.GatherScatterMode.PROMISE_IN_BOUNDS)

        def scale(b, k):
            buf = bufs[k]

            @pl.loop(0, eb, step=_L)
            def _(j):
                w16 = ewv[b, pl.ds(j, _L)]
                for l in range(_L):
                    wb = lane_bcast(w16, l)
                    for f0 in range(0, h, _L):
                        sl = (j + l, pl.ds(f0, _L))
                        buf[sl] = buf[sl] * wb

        def pipeline(row_hbm, col_hbm, ew_hbm, nb):
            pltpu.sync_copy(row_hbm.at[sid], rowv.at[pl.ds(0, nb)])
            pltpu.sync_copy(col_hbm.at[sid], colv.at[pl.ds(0, nb)])
            pltpu.sync_copy(ew_hbm.at[sid], ewv.at[pl.ds(0, nb)])
            start_gather(0, 0)

            @pl.loop(0, nb, step=2)
            def _(b0):
                wait_gather(b0, 0)

                @pl.when(b0 > 0)
                def _():
                    wait_scatter(b0 - 1, 1)

                start_gather(b0 + 1, 1)
                scale(b0, 0)
                start_scatter(b0, 0)
                wait_gather(b0 + 1, 1)
                scale(b0 + 1, 1)

                @pl.when(b0 + 2 < nb)
                def _():
                    wait_scatter(b0, 0)
                    start_gather(b0 + 2, 0)

                start_scatter(b0 + 1, 1)

            wait_scatter(nb - 2, 0)
            wait_scatter(nb - 1, 1)

        @pl.when(cid == 0)
        def _():
            pipeline(r0_hbm, c0_hbm, w0_hbm, nb0)

        @pl.when(cid == 1)
        def _():
            pipeline(r1_hbm, c1_hbm, w1_hbm, nb1)

        plsc.subcore_barrier()
        pltpu.sync_copy(acc.at[pl.ds(base, rpt)],
                        out_hbm.at[cid, pl.ds(base, rpt)])

    return agg_kernel


def _tc_h2(x, w1, b1, w2):
    n, f_in = x.shape
    hdim = w1.shape[1]
    blk = 1000

    def body(x_ref, w1_ref, b1_ref, w2_ref, o_ref):
        hb = jnp.dot(x_ref[...], w1_ref[...],
                     preferred_element_type=jnp.float32) + b1_ref[...]
        hb = jnp.maximum(hb, 0.0)
        o_ref[...] = jnp.dot(hb, w2_ref[...],
                             preferred_element_type=jnp.float32)

    return pl.pallas_call(
        body,
        grid=(n // blk,),
        in_specs=[
            pl.BlockSpec((blk, f_in), lambda i: (i, 0)),
            pl.BlockSpec((f_in, hdim), lambda i: (0, 0)),
            pl.BlockSpec((1, hdim), lambda i: (0, 0)),
            pl.BlockSpec((hdim, hdim), lambda i: (0, 0)),
        ],
        out_specs=pl.BlockSpec((blk, hdim), lambda i: (i, 0)),
        out_shape=jax.ShapeDtypeStruct((n, hdim), jnp.float32),
    )(x, w1, b1.reshape(1, hdim), w2)


def _dinv_col(d_ref):
    s = jnp.sum(d_ref[...], axis=1, keepdims=True)
    return 1.0 / jnp.sqrt(1.0 + s)


def _tc_g(deg_t, h2):
    n, nw = deg_t.shape
    hdim = h2.shape[1]
    blk = 1000

    def body(d_ref, h_ref, o_ref):
        o_ref[...] = _dinv_col(d_ref) * h_ref[...]

    return pl.pallas_call(
        body,
        grid=(n // blk,),
        in_specs=[
            pl.BlockSpec((blk, nw), lambda i: (i, 0)),
            pl.BlockSpec((blk, hdim), lambda i: (i, 0)),
        ],
        out_specs=pl.BlockSpec((blk, hdim), lambda i: (i, 0)),
        out_shape=jax.ShapeDtypeStruct((n, hdim), jnp.float32),
    )(deg_t, h2)


def _tc_final(deg_t, g, parts, b_conv, w_out, b_out):
    n, nw = deg_t.shape
    hdim = g.shape[1]
    c = w_out.shape[1]
    blk = 1000

    def body(d_ref, g_ref, p_ref, bc_ref, wo_ref, bo_ref, o_ref):
        dinv = _dinv_col(d_ref)
        agg = p_ref[0] + p_ref[1] - g_ref[...]
        hr = jnp.maximum(dinv * agg + bc_ref[...], 0.0)
        logits = jnp.dot(hr, wo_ref[...],
                         preferred_element_type=jnp.float32) + bo_ref[...]
        m = jnp.max(logits, axis=-1, keepdims=True)
        ex = jnp.exp(logits - m)
        lse = jnp.log(jnp.sum(ex, axis=-1, keepdims=True))
        o_ref[...] = (logits - m) - lse

    return pl.pallas_call(
        body,
        grid=(n // blk,),
        in_specs=[
            pl.BlockSpec((blk, nw), lambda i: (i, 0)),
            pl.BlockSpec((blk, hdim), lambda i: (i, 0)),
            pl.BlockSpec((_NC, blk, hdim), lambda i: (0, i, 0)),
            pl.BlockSpec((1, hdim), lambda i: (0, 0)),
            pl.BlockSpec((hdim, c), lambda i: (0, 0)),
            pl.BlockSpec((1, c), lambda i: (0, 0)),
        ],
        out_specs=pl.BlockSpec((blk, c), lambda i: (i, 0)),
        out_shape=jax.ShapeDtypeStruct((n, c), jnp.float32),
    )(deg_t, g, parts, b_conv.reshape(1, hdim), w_out,
      b_out.reshape(1, c))


def kernel(x, edge_index, edge_weight,
           W_first, b_first, W_conv, b_conv, W_out, b_out):
    n, _ = x.shape
    hdim = W_conv.shape[0]
    e = edge_weight.shape[0]
    nbt = 2 * (-(-e // (2 * _NS * _EB)))
    ep = _NS * nbt * _EB
    pad = ep - e

    row = edge_index[0]
    col = edge_index[1]
    ew = edge_weight
    if pad:
        zi = jnp.zeros((pad,), row.dtype)
        row = jnp.concatenate([row, zi])
        col = jnp.concatenate([col, zi])
        ew = jnp.concatenate([ew, jnp.zeros((pad,), ew.dtype)])

    nb0 = max(2, 2 * int(round(nbt * 0.625 / 2)))
    nb1 = nbt - nb0
    s0 = _NS * nb0 * _EB
    r0 = row[:s0].reshape(_NS, nb0, _EB)
    c0 = col[:s0].reshape(_NS, nb0, _EB)
    w0 = ew[:s0].reshape(_NS, nb0, _EB)
    r1 = row[s0:].reshape(_NS, nb1, _EB)
    c1 = col[s0:].reshape(_NS, nb1, _EB)
    w1 = ew[s0:].reshape(_NS, nb1, _EB)

    eper = ep // _NW
    deg_parts = _make_deg_kernel(n, eper)(
        col.reshape(_NW, 1, eper), ew.reshape(_NW, 1, eper))
    h2 = _tc_h2(x, W_first, b_first, W_conv)
    deg_t = deg_parts.reshape(_NW, n).T
    g = _tc_g(deg_t, h2)
    n_pad = -(-n // (8 * _NS)) * (8 * _NS)
    g_p = (g if n_pad == n else
           jnp.concatenate([g, jnp.zeros((n_pad - n, hdim), g.dtype)]))
    parts = _make_agg_kernel(n_pad, hdim, nb0, nb1, _EB)(
        g_p, r0, c0, w0, r1, c1, w1)
    return _tc_final(deg_t, g, parts, b_conv, W_out, b_out)


# speedup vs baseline: 1.8436x; 1.1363x over previous
import dataclasses
import functools

import jax
import jax.numpy as jnp
from jax import lax
from jax.experimental import pallas as pl
from jax.experimental.pallas import tpu as pltpu
from jax.experimental.pallas import tpu_sc as plsc

_NC = 2
_NS = 16
_NW = _NC * _NS
_L = 16
_EB = 320


def _vector_mesh():
    return plsc.VectorSubcoreMesh(
        core_axis_name="c", subcore_axis_name="s",
        num_cores=_NC, num_subcores=_NS)


def _sc_params():
    cp = pltpu.CompilerParams()
    fields = pltpu.CompilerParams.__dataclass_fields__
    if "needs_layout_passes" in fields:
        cp = dataclasses.replace(cp, needs_layout_passes=False)
    if "use_tc_tiling_on_sc" in fields:
        cp = dataclasses.replace(cp, use_tc_tiling_on_sc=False)
    return cp


def _make_deg_kernel(n, eper):
    @functools.partial(
        pl.kernel,
        out_type=jax.ShapeDtypeStruct((_NW, 1, n), jnp.float32),
        mesh=_vector_mesh(),
        compiler_params=_sc_params(),
        scratch_types=[
            pltpu.VMEM((1, eper), jnp.int32),
            pltpu.VMEM((1, eper), jnp.float32),
            pltpu.VMEM((1, n), jnp.float32),
        ],
    )
    def deg_kernel(col_hbm, ew_hbm, out_hbm, colv, ewv, hist):
        cid = lax.axis_index("c")
        sid = lax.axis_index("s")
        wid = sid * _NC + cid
        pltpu.sync_copy(col_hbm.at[wid], colv)
        pltpu.sync_copy(ew_hbm.at[wid], ewv)
        zeros = jnp.zeros((_L,), jnp.float32)
        zi = jnp.zeros((_L,), jnp.int32)

        @pl.loop(0, n, step=_L)
        def _(i):
            hist[0, pl.ds(i, _L)] = zeros

        @pl.loop(0, eper, step=_L)
        def _(j):
            idx = colv[0, pl.ds(j, _L)]
            w = ewv[0, pl.ds(j, _L)]
            plsc.addupdate_scatter(hist, [zi, idx], w)

        pltpu.sync_copy(hist, out_hbm.at[wid])

    return deg_kernel


def _make_agg_kernel(n, h, nb0, nb1, eb):
    rpt = n // _NS
    nbm = max(nb0, nb1)

    assert nb0 % 2 == 0 and nb1 % 2 == 0

    @functools.partial(
        pl.kernel,
        out_type=jax.ShapeDtypeStruct((_NC, n, h), jnp.float32),
        mesh=_vector_mesh(),
        compiler_params=_sc_params(),
        scratch_types=[
            pltpu.VMEM_SHARED((n, h), jnp.float32),
            pltpu.VMEM((nbm, eb), jnp.int32),
            pltpu.VMEM((nbm, eb), jnp.int32),
            pltpu.VMEM((nbm, eb), jnp.float32),
            pltpu.VMEM((eb, h), jnp.float32),
            pltpu.VMEM((eb, h), jnp.float32),
            pltpu.SemaphoreType.DMA,
            pltpu.SemaphoreType.DMA,
            pltpu.SemaphoreType.DMA,
            pltpu.SemaphoreType.DMA,
        ],
    )
    def agg_kernel(g_hbm, r0_hbm, c0_hbm, w0_hbm, r1_hbm, c1_hbm, w1_hbm,
                   out_hbm, acc, rowv, colv, ewv, rows0, rows1,
                   gsem0, gsem1, ssem0, ssem1):
        cid = lax.axis_index("c")
        sid = lax.axis_index("s")
        base = pl.multiple_of(sid * rpt, 8)
        pltpu.sync_copy(g_hbm.at[pl.ds(base, rpt)], acc.at[pl.ds(base, rpt)])
        plsc.subcore_barrier()

        bufs = (rows0, rows1)
        gsems = (gsem0, gsem1)
        ssems = (ssem0, ssem1)

        def start_gather(b, k):
            pltpu.async_copy(g_hbm.at[rowv.at[b]], bufs[k], gsems[k])

        def wait_gather(b, k):
            pltpu.make_async_copy(g_hbm.at[rowv.at[b]], bufs[k],
                                  gsems[k]).wait()

        def start_scatter(b, k):
            pltpu.async_copy(bufs[k], acc.at[colv.at[b]], ssems[k], add=True)

        def wait_scatter(b, k):
            pltpu.make_async_copy(bufs[k], acc.at[colv.at[b]],
                                  ssems[k]).wait()

        bcast_dn = lax.GatherDimensionNumbers(
            offset_dims=(), collapsed_slice_dims=(0,), start_index_map=(0,))

        def lane_bcast(v16, l):
            idx = jnp.full((_L, 1), l, jnp.int32)
            return lax.gather(v16, idx, bcast_dn, (1,),
                              mode=lax.GatherScatterMode.PROMISE_IN_BOUNDS)

        def scale(b, k):
            buf = bufs[k]

            @pl.loop(0, eb, step=_L, unroll=2)
            def _(j):
                w16 = ewv[b, pl.ds(j, _L)]
                for l in range(_L):
                    wb = lane_bcast(w16, l)
                    for f0 in range(0, h, _L):
                        sl = (j + l, pl.ds(f0, _L))
                        buf[sl] = buf[sl] * wb

        def pipeline(row_hbm, col_hbm, ew_hbm, nb):
            pltpu.sync_copy(row_hbm.at[sid], rowv.at[pl.ds(0, nb)])
            pltpu.sync_copy(col_hbm.at[sid], colv.at[pl.ds(0, nb)])
            pltpu.sync_copy(ew_hbm.at[sid], ewv.at[pl.ds(0, nb)])
            start_gather(0, 0)

            @pl.loop(0, nb, step=2)
            def _(b0):
                wait_gather(b0, 0)

                @pl.when(b0 > 0)
                def _():
                    wait_scatter(b0 - 1, 1)

                start_gather(b0 + 1, 1)
                scale(b0, 0)
                start_scatter(b0, 0)
                wait_gather(b0 + 1, 1)
                scale(b0 + 1, 1)

                @pl.when(b0 + 2 < nb)
                def _():
                    wait_scatter(b0, 0)
                    start_gather(b0 + 2, 0)

                start_scatter(b0 + 1, 1)

            wait_scatter(nb - 2, 0)
            wait_scatter(nb - 1, 1)

        @pl.when(cid == 0)
        def _():
            pipeline(r0_hbm, c0_hbm, w0_hbm, nb0)

        @pl.when(cid == 1)
        def _():
            pipeline(r1_hbm, c1_hbm, w1_hbm, nb1)

        plsc.subcore_barrier()
        pltpu.sync_copy(acc.at[pl.ds(base, rpt)],
                        out_hbm.at[cid, pl.ds(base, rpt)])

    return agg_kernel


def _tc_h2(x, w1, b1, w2):
    n, f_in = x.shape
    hdim = w1.shape[1]
    blk = 1000

    def body(x_ref, w1_ref, b1_ref, w2_ref, o_ref):
        hb = jnp.dot(x_ref[...], w1_ref[...],
                     preferred_element_type=jnp.float32) + b1_ref[...]
        hb = jnp.maximum(hb, 0.0)
        o_ref[...] = jnp.dot(hb, w2_ref[...],
                             preferred_element_type=jnp.float32)

    return pl.pallas_call(
        body,
        grid=(n // blk,),
        in_specs=[
            pl.BlockSpec((blk, f_in), lambda i: (i, 0)),
            pl.BlockSpec((f_in, hdim), lambda i: (0, 0)),
            pl.BlockSpec((1, hdim), lambda i: (0, 0)),
            pl.BlockSpec((hdim, hdim), lambda i: (0, 0)),
        ],
        out_specs=pl.BlockSpec((blk, hdim), lambda i: (i, 0)),
        out_shape=jax.ShapeDtypeStruct((n, hdim), jnp.float32),
    )(x, w1, b1.reshape(1, hdim), w2)


def _dinv_col(d_ref):
    s = jnp.sum(d_ref[...], axis=1, keepdims=True)
    return 1.0 / jnp.sqrt(1.0 + s)


def _tc_g(deg_t, h2):
    n, nw = deg_t.shape
    hdim = h2.shape[1]
    blk = 1000

    def body(d_ref, h_ref, o_ref):
        o_ref[...] = _dinv_col(d_ref) * h_ref[...]

    return pl.pallas_call(
        body,
        grid=(n // blk,),
        in_specs=[
            pl.BlockSpec((blk, nw), lambda i: (i, 0)),
            pl.BlockSpec((blk, hdim), lambda i: (i, 0)),
        ],
        out_specs=pl.BlockSpec((blk, hdim), lambda i: (i, 0)),
        out_shape=jax.ShapeDtypeStruct((n, hdim), jnp.float32),
    )(deg_t, h2)


def _tc_final(deg_t, g, parts, b_conv, w_out, b_out):
    n, nw = deg_t.shape
    hdim = g.shape[1]
    c = w_out.shape[1]
    blk = 1000

    def body(d_ref, g_ref, p_ref, bc_ref, wo_ref, bo_ref, o_ref):
        dinv = _dinv_col(d_ref)
        agg = p_ref[0] + p_ref[1] - g_ref[...]
        hr = jnp.maximum(dinv * agg + bc_ref[...], 0.0)
        logits = jnp.dot(hr, wo_ref[...],
                         preferred_element_type=jnp.float32) + bo_ref[...]
        m = jnp.max(logits, axis=-1, keepdims=True)
        ex = jnp.exp(logits - m)
        lse = jnp.log(jnp.sum(ex, axis=-1, keepdims=True))
        o_ref[...] = (logits - m) - lse

    return pl.pallas_call(
        body,
        grid=(n // blk,),
        in_specs=[
            pl.BlockSpec((blk, nw), lambda i: (i, 0)),
            pl.BlockSpec((blk, hdim), lambda i: (i, 0)),
            pl.BlockSpec((_NC, blk, hdim), lambda i: (0, i, 0)),
            pl.BlockSpec((1, hdim), lambda i: (0, 0)),
            pl.BlockSpec((hdim, c), lambda i: (0, 0)),
            pl.BlockSpec((1, c), lambda i: (0, 0)),
        ],
        out_specs=pl.BlockSpec((blk, c), lambda i: (i, 0)),
        out_shape=jax.ShapeDtypeStruct((n, c), jnp.float32),
    )(deg_t, g, parts, b_conv.reshape(1, hdim), w_out,
      b_out.reshape(1, c))


def kernel(x, edge_index, edge_weight,
           W_first, b_first, W_conv, b_conv, W_out, b_out):
    n, _ = x.shape
    hdim = W_conv.shape[0]
    e = edge_weight.shape[0]
    nbt = 2 * (-(-e // (2 * _NS * _EB)))
    ep = _NS * nbt * _EB
    pad = ep - e

    row = edge_index[0]
    col = edge_index[1]
    ew = edge_weight
    if pad:
        zi = jnp.zeros((pad,), row.dtype)
        row = jnp.concatenate([row, zi])
        col = jnp.concatenate([col, zi])
        ew = jnp.concatenate([ew, jnp.zeros((pad,), ew.dtype)])

    nb0 = max(2, 2 * int(round(nbt * 0.656 / 2)))
    nb1 = nbt - nb0
    s0 = _NS * nb0 * _EB
    r0 = row[:s0].reshape(_NS, nb0, _EB)
    c0 = col[:s0].reshape(_NS, nb0, _EB)
    w0 = ew[:s0].reshape(_NS, nb0, _EB)
    r1 = row[s0:].reshape(_NS, nb1, _EB)
    c1 = col[s0:].reshape(_NS, nb1, _EB)
    w1 = ew[s0:].reshape(_NS, nb1, _EB)

    eper = ep // _NW
    deg_parts = _make_deg_kernel(n, eper)(
        col.reshape(_NW, 1, eper), ew.reshape(_NW, 1, eper))
    h2 = _tc_h2(x, W_first, b_first, W_conv)
    deg_t = deg_parts.reshape(_NW, n).T
    g = _tc_g(deg_t, h2)
    n_pad = -(-n // (8 * _NS)) * (8 * _NS)
    g_p = (g if n_pad == n else
           jnp.concatenate([g, jnp.zeros((n_pad - n, hdim), g.dtype)]))
    parts = _make_agg_kernel(n_pad, hdim, nb0, nb1, _EB)(
        g_p, r0, c0, w0, r1, c1, w1)
    return _tc_final(deg_t, g, parts, b_conv, W_out, b_out)


# speedup vs baseline: 1.9461x; 1.0556x over previous
import dataclasses
import functools

import jax
import jax.numpy as jnp
from jax import lax
from jax.experimental import pallas as pl
from jax.experimental.pallas import tpu as pltpu
from jax.experimental.pallas import tpu_sc as plsc

_NC = 2
_NS = 16
_NW = _NC * _NS
_L = 16
_EB = 320


def _vector_mesh():
    return plsc.VectorSubcoreMesh(
        core_axis_name="c", subcore_axis_name="s",
        num_cores=_NC, num_subcores=_NS)


def _sc_params():
    cp = pltpu.CompilerParams()
    fields = pltpu.CompilerParams.__dataclass_fields__
    if "needs_layout_passes" in fields:
        cp = dataclasses.replace(cp, needs_layout_passes=False)
    if "use_tc_tiling_on_sc" in fields:
        cp = dataclasses.replace(cp, use_tc_tiling_on_sc=False)
    return cp


def _make_deg_kernel(n, eper):
    @functools.partial(
        pl.kernel,
        out_type=jax.ShapeDtypeStruct((_NW, 1, n), jnp.float32),
        mesh=_vector_mesh(),
        compiler_params=_sc_params(),
        scratch_types=[
            pltpu.VMEM((1, eper), jnp.int32),
            pltpu.VMEM((1, eper), jnp.float32),
            pltpu.VMEM((1, n), jnp.float32),
        ],
    )
    def deg_kernel(col_hbm, ew_hbm, out_hbm, colv, ewv, hist):
        cid = lax.axis_index("c")
        sid = lax.axis_index("s")
        wid = sid * _NC + cid
        pltpu.sync_copy(col_hbm.at[wid], colv)
        pltpu.sync_copy(ew_hbm.at[wid], ewv)
        zeros = jnp.zeros((_L,), jnp.float32)
        zi = jnp.zeros((_L,), jnp.int32)

        @pl.loop(0, n, step=_L)
        def _(i):
            hist[0, pl.ds(i, _L)] = zeros

        @pl.loop(0, eper, step=_L)
        def _(j):
            idx = colv[0, pl.ds(j, _L)]
            w = ewv[0, pl.ds(j, _L)]
            plsc.addupdate_scatter(hist, [zi, idx], w)

        pltpu.sync_copy(hist, out_hbm.at[wid])

    return deg_kernel


def _make_agg_kernel(n, h, nb0, nb1, eb):
    rpt = n // _NS
    nbm = max(nb0, nb1)

    assert nb0 % 2 == 0 and nb1 % 2 == 0

    @functools.partial(
        pl.kernel,
        out_type=jax.ShapeDtypeStruct((_NC, n, h), jnp.float32),
        mesh=_vector_mesh(),
        compiler_params=_sc_params(),
        scratch_types=[
            pltpu.VMEM_SHARED((n, h), jnp.float32),
            pltpu.VMEM((nbm, eb), jnp.int32),
            pltpu.VMEM((nbm, eb), jnp.int32),
            pltpu.VMEM((nbm, eb), jnp.float32),
            pltpu.VMEM((eb, h), jnp.float32),
            pltpu.VMEM((eb, h), jnp.float32),
            pltpu.SemaphoreType.DMA,
            pltpu.SemaphoreType.DMA,
            pltpu.SemaphoreType.DMA,
            pltpu.SemaphoreType.DMA,
        ],
    )
    def agg_kernel(g_hbm, r0_hbm, c0_hbm, w0_hbm, r1_hbm, c1_hbm, w1_hbm,
                   out_hbm, acc, rowv, colv, ewv, rows0, rows1,
                   gsem0, gsem1, ssem0, ssem1):
        cid = lax.axis_index("c")
        sid = lax.axis_index("s")
        base = pl.multiple_of(sid * rpt, 8)
        pltpu.sync_copy(g_hbm.at[pl.ds(base, rpt)], acc.at[pl.ds(base, rpt)])
        plsc.subcore_barrier()

        bufs = (rows0, rows1)
        gsems = (gsem0, gsem1)
        ssems = (ssem0, ssem1)

        def start_gather(b, k):
            pltpu.async_copy(g_hbm.at[rowv.at[b]], bufs[k], gsems[k])

        def wait_gather(b, k):
            pltpu.make_async_copy(g_hbm.at[rowv.at[b]], bufs[k],
                                  gsems[k]).wait()

        def start_scatter(b, k):
            pltpu.async_copy(bufs[k], acc.at[colv.at[b]], ssems[k], add=True)

        def wait_scatter(b, k):
            pltpu.make_async_copy(bufs[k], acc.at[colv.at[b]],
                                  ssems[k]).wait()

        bcast_dn = lax.GatherDimensionNumbers(
            offset_dims=(), collapsed_slice_dims=(0,), start_index_map=(0,))

        def lane_bcast(v16, l):
            idx = jnp.full((_L, 1), l, jnp.int32)
            return lax.gather(v16, idx, bcast_dn, (1,),
                              mode=lax.GatherScatterMode.PROMISE_IN_BOUNDS)

        def scale(b, k):
            buf = bufs[k]

            @pl.loop(0, eb, step=_L, unroll=4)
            def _(j):
                w16 = ewv[b, pl.ds(j, _L)]
                for l in range(_L):
                    wb = lane_bcast(w16, l)
                    for f0 in range(0, h, _L):
                        sl = (j + l, pl.ds(f0, _L))
                        buf[sl] = buf[sl] * wb

        def pipeline(row_hbm, col_hbm, ew_hbm, nb):
            pltpu.sync_copy(row_hbm.at[sid], rowv.at[pl.ds(0, nb)])
            pltpu.sync_copy(col_hbm.at[sid], colv.at[pl.ds(0, nb)])
            pltpu.sync_copy(ew_hbm.at[sid], ewv.at[pl.ds(0, nb)])
            start_gather(0, 0)

            @pl.loop(0, nb, step=2)
            def _(b0):
                wait_gather(b0, 0)

                @pl.when(b0 > 0)
                def _():
                    wait_scatter(b0 - 1, 1)

                start_gather(b0 + 1, 1)
                scale(b0, 0)
                start_scatter(b0, 0)
                wait_gather(b0 + 1, 1)
                scale(b0 + 1, 1)

                @pl.when(b0 + 2 < nb)
                def _():
                    wait_scatter(b0, 0)
                    start_gather(b0 + 2, 0)

                start_scatter(b0 + 1, 1)

            wait_scatter(nb - 2, 0)
            wait_scatter(nb - 1, 1)

        @pl.when(cid == 0)
        def _():
            pipeline(r0_hbm, c0_hbm, w0_hbm, nb0)

        @pl.when(cid == 1)
        def _():
            pipeline(r1_hbm, c1_hbm, w1_hbm, nb1)

        plsc.subcore_barrier()
        pltpu.sync_copy(acc.at[pl.ds(base, rpt)],
                        out_hbm.at[cid, pl.ds(base, rpt)])

    return agg_kernel


def _tc_h2(x, w1, b1, w2):
    n, f_in = x.shape
    hdim = w1.shape[1]
    blk = 1000

    def body(x_ref, w1_ref, b1_ref, w2_ref, o_ref):
        hb = jnp.dot(x_ref[...], w1_ref[...],
                     preferred_element_type=jnp.float32) + b1_ref[...]
        hb = jnp.maximum(hb, 0.0)
        o_ref[...] = jnp.dot(hb, w2_ref[...],
                             preferred_element_type=jnp.float32)

    return pl.pallas_call(
        body,
        grid=(n // blk,),
        in_specs=[
            pl.BlockSpec((blk, f_in), lambda i: (i, 0)),
            pl.BlockSpec((f_in, hdim), lambda i: (0, 0)),
            pl.BlockSpec((1, hdim), lambda i: (0, 0)),
            pl.BlockSpec((hdim, hdim), lambda i: (0, 0)),
        ],
        out_specs=pl.BlockSpec((blk, hdim), lambda i: (i, 0)),
        out_shape=jax.ShapeDtypeStruct((n, hdim), jnp.float32),
    )(x, w1, b1.reshape(1, hdim), w2)


def _dinv_col(d_ref):
    s = jnp.sum(d_ref[...], axis=1, keepdims=True)
    return 1.0 / jnp.sqrt(1.0 + s)


def _tc_g(deg_t, h2):
    n, nw = deg_t.shape
    hdim = h2.shape[1]
    blk = 1000

    def body(d_ref, h_ref, o_ref):
        o_ref[...] = _dinv_col(d_ref) * h_ref[...]

    return pl.pallas_call(
        body,
        grid=(n // blk,),
        in_specs=[
            pl.BlockSpec((blk, nw), lambda i: (i, 0)),
            pl.BlockSpec((blk, hdim), lambda i: (i, 0)),
        ],
        out_specs=pl.BlockSpec((blk, hdim), lambda i: (i, 0)),
        out_shape=jax.ShapeDtypeStruct((n, hdim), jnp.float32),
    )(deg_t, h2)


def _tc_final(deg_t, g, parts, b_conv, w_out, b_out):
    n, nw = deg_t.shape
    hdim = g.shape[1]
    c = w_out.shape[1]
    blk = 1000

    def body(d_ref, g_ref, p_ref, bc_ref, wo_ref, bo_ref, o_ref):
        dinv = _dinv_col(d_ref)
        agg = p_ref[0] + p_ref[1] - g_ref[...]
        hr = jnp.maximum(dinv * agg + bc_ref[...], 0.0)
        logits = jnp.dot(hr, wo_ref[...],
                         preferred_element_type=jnp.float32) + bo_ref[...]
        m = jnp.max(logits, axis=-1, keepdims=True)
        ex = jnp.exp(logits - m)
        lse = jnp.log(jnp.sum(ex, axis=-1, keepdims=True))
        o_ref[...] = (logits - m) - lse

    return pl.pallas_call(
        body,
        grid=(n // blk,),
        in_specs=[
            pl.BlockSpec((blk, nw), lambda i: (i, 0)),
            pl.BlockSpec((blk, hdim), lambda i: (i, 0)),
            pl.BlockSpec((_NC, blk, hdim), lambda i: (0, i, 0)),
            pl.BlockSpec((1, hdim), lambda i: (0, 0)),
            pl.BlockSpec((hdim, c), lambda i: (0, 0)),
            pl.BlockSpec((1, c), lambda i: (0, 0)),
        ],
        out_specs=pl.BlockSpec((blk, c), lambda i: (i, 0)),
        out_shape=jax.ShapeDtypeStruct((n, c), jnp.float32),
    )(deg_t, g, parts, b_conv.reshape(1, hdim), w_out,
      b_out.reshape(1, c))


def kernel(x, edge_index, edge_weight,
           W_first, b_first, W_conv, b_conv, W_out, b_out):
    n, _ = x.shape
    hdim = W_conv.shape[0]
    e = edge_weight.shape[0]
    nbt = 2 * (-(-e // (2 * _NS * _EB)))
    ep = _NS * nbt * _EB
    pad = ep - e

    row = edge_index[0]
    col = edge_index[1]
    ew = edge_weight
    if pad:
        zi = jnp.zeros((pad,), row.dtype)
        row = jnp.concatenate([row, zi])
        col = jnp.concatenate([col, zi])
        ew = jnp.concatenate([ew, jnp.zeros((pad,), ew.dtype)])

    nb0 = max(2, 2 * int(round(nbt * 0.78 / 2)))
    nb1 = nbt - nb0
    s0 = _NS * nb0 * _EB
    r0 = row[:s0].reshape(_NS, nb0, _EB)
    c0 = col[:s0].reshape(_NS, nb0, _EB)
    w0 = ew[:s0].reshape(_NS, nb0, _EB)
    r1 = row[s0:].reshape(_NS, nb1, _EB)
    c1 = col[s0:].reshape(_NS, nb1, _EB)
    w1 = ew[s0:].reshape(_NS, nb1, _EB)

    eper = ep // _NW
    deg_parts = _make_deg_kernel(n, eper)(
        col.reshape(_NW, 1, eper), ew.reshape(_NW, 1, eper))
    h2 = _tc_h2(x, W_first, b_first, W_conv)
    deg_t = deg_parts.reshape(_NW, n).T
    g = _tc_g(deg_t, h2)
    n_pad = -(-n // (8 * _NS)) * (8 * _NS)
    g_p = (g if n_pad == n else
           jnp.concatenate([g, jnp.zeros((n_pad - n, hdim), g.dtype)]))
    parts = _make_agg_kernel(n_pad, hdim, nb0, nb1, _EB)(
        g_p, r0, c0, w0, r1, c1, w1)
    return _tc_final(deg_t, g, parts, b_conv, W_out, b_out)
